# baseline (speedup 1.0000x reference)
import functools

import jax
import jax.numpy as jnp
from jax import lax
from jax.experimental import pallas as pl
from jax.experimental.pallas import tpu as pltpu
from jax.experimental.pallas import tpu_sc as plsc

NE = 1000000
DIM = 32
NC = 2
NS = 16
L = 16
NW = NC * NS

BLK = 512
FULL_E = (NE // 128) * 128
NBLK = FULL_E // BLK
WPT = NW // 4
TAIL = NE - FULL_E

C = 128


def _detile_one(t_hbm, s_hbm, o_hbm, bslot, in_v, out_v, tl_v, sem):
    nper = NBLK // WPT + 1

    def blk_body(j, carry):
        bidx = bslot + j * WPT

        @pl.when(bidx < NBLK)
        def _():
            e0 = pl.multiple_of(bidx * BLK, 128)
            pltpu.sync_copy(t_hbm.at[:, pl.ds(e0, BLK)], in_v)

            def row_body(e, c2):
                for d0 in range(0, DIM, L):
                    rows = d0 + lax.iota(jnp.int32, L)
                    cols = jnp.full((L,), 0, jnp.int32) + e
                    out_v[e, pl.ds(d0, L)] = plsc.load_gather(
                        in_v, [rows, cols])
                return c2

            lax.fori_loop(0, BLK, row_body, 0)
            pltpu.sync_copy(out_v, o_hbm.at[pl.ds(e0, BLK), :])

        return carry

    lax.fori_loop(0, nper, blk_body, 0)

    @pl.when(bslot == 0)
    def _():
        pltpu.sync_copy(s_hbm, tl_v)

        def row_body(e, c2):
            for d0 in range(0, DIM, L):
                rows = d0 + lax.iota(jnp.int32, L)
                cols = jnp.full((L,), 0, jnp.int32) + e
                out_v[e, pl.ds(d0, L)] = plsc.load_gather(tl_v, [rows, cols])
            return c2

        lax.fori_loop(0, TAIL, row_body, 0)
        pltpu.sync_copy(out_v.at[pl.ds(0, TAIL), :],
                        o_hbm.at[pl.ds(FULL_E, TAIL), :])


def _make_detile():
    mesh = plsc.VectorSubcoreMesh(core_axis_name="c", subcore_axis_name="s")
    tbl = jax.ShapeDtypeStruct((NE, DIM), jnp.float32)

    @functools.partial(
        pl.kernel,
        out_type=(tbl, tbl, tbl, tbl),
        mesh=mesh,
        compiler_params=pltpu.CompilerParams(
            needs_layout_passes=False, use_tc_tiling_on_sc=True),
        scratch_types=[
            pltpu.VMEM((DIM, BLK), jnp.float32),
            pltpu.VMEM((BLK, DIM), jnp.float32),
            pltpu.VMEM((DIM, TAIL), jnp.float32),
            pltpu.SemaphoreType.DMA,
        ],
    )
    def k(t0, t1, t2, t3, s0, s1, s2, s3, o0, o1, o2, o3,
          in_v, out_v, tl_v, sem):
        wid = lax.axis_index("s") * NC + lax.axis_index("c")
        tid = wid // WPT
        bslot = wid % WPT
        for i, (t, s, o) in enumerate(
                ((t0, s0, o0), (t1, s1, o1), (t2, s2, o2), (t3, s3, o3))):
            @pl.when(tid == i)
            def _(t=t, s=s, o=o):
                _detile_one(t, s, o, bslot, in_v, out_v, tl_v, sem)

    return k


def _make_score(tot):
    per_w = tot // NW
    n_chunks = per_w // C
    mesh = plsc.VectorSubcoreMesh(core_axis_name="c", subcore_axis_name="s")

    @functools.partial(
        pl.kernel,
        out_type=jax.ShapeDtypeStruct((tot,), jnp.float32),
        mesh=mesh,
        compiler_params=pltpu.CompilerParams(
            needs_layout_passes=False, use_tc_tiling_on_sc=False),
        scratch_types=[
            pltpu.VMEM((C,), jnp.int32),
            pltpu.VMEM((C,), jnp.int32),
            pltpu.VMEM((C,), jnp.int32),
            pltpu.VMEM((C, DIM), jnp.float32),
            pltpu.VMEM((C, DIM), jnp.float32),
            pltpu.VMEM((C, DIM), jnp.float32),
            pltpu.VMEM((C, DIM), jnp.float32),
            pltpu.VMEM((C, DIM), jnp.float32),
            pltpu.VMEM((C, DIM), jnp.float32),
            pltpu.VMEM((C,), jnp.float32),
            pltpu.SemaphoreType.DMA,
        ],
    )
    def k(h_hbm, r_hbm, t_hbm, ere_hbm, eim_hbm, rre_hbm, rim_hbm, out_hbm,
          h_v, r_v, t_v, hre, him, tre, tim, rre, rim, sc_v, sem):
        wid = lax.axis_index("s") * NC + lax.axis_index("c")
        w_base = wid * per_w

        def chunk_body(ci, carry):
            base = w_base + ci * C
            pltpu.sync_copy(h_hbm.at[pl.ds(base, C)], h_v)
            pltpu.sync_copy(r_hbm.at[pl.ds(base, C)], r_v)
            pltpu.sync_copy(t_hbm.at[pl.ds(base, C)], t_v)
            cps = [
                pltpu.async_copy(ere_hbm.at[h_v], hre, sem),
                pltpu.async_copy(eim_hbm.at[h_v], him, sem),
                pltpu.async_copy(ere_hbm.at[t_v], tre, sem),
                pltpu.async_copy(eim_hbm.at[t_v], tim, sem),
                pltpu.async_copy(rre_hbm.at[r_v], rre, sem),
                pltpu.async_copy(rim_hbm.at[r_v], rim, sem),
            ]
            for cp in cps:
                cp.wait()

            def group_body(g, gcarry):
                row = g * L + lax.iota(jnp.int32, L)
                acc = jnp.zeros((L,), jnp.float32)
                for d in range(DIM):
                    col = jnp.full((L,), d, jnp.int32)
                    a = plsc.load_gather(hre, [row, col])
                    b = plsc.load_gather(him, [row, col])
                    u = plsc.load_gather(tre, [row, col])
                    v = plsc.load_gather(tim, [row, col])
                    p = plsc.load_gather(rre, [row, col])
                    q = plsc.load_gather(rim, [row, col])
                    acc = acc + p * (a * u + b * v) + q * (a * v - b * u)
                sc_v[pl.ds(g * L, L)] = acc
                return gcarry

            lax.fori_loop(0, C // L, group_body, 0)
            pltpu.sync_copy(sc_v, out_hbm.at[pl.ds(base, C)])
            return carry

        lax.fori_loop(0, n_chunks, chunk_body, 0)

    return k


@functools.partial(jax.jit, static_argnames=("tot",))
def _complex_scores(h, r, t, ent_re, ent_im, rel_re, rel_im, tot):
    tables = (ent_re, ent_im, rel_re, rel_im)
    tviews = [x.T for x in tables]
    sides = [v[:, FULL_E:] for v in tviews]
    dre, dim_, drre, drim = _make_detile()(*tviews, *sides)
    return _make_score(tot)(h, r, t, dre, dim_, drre, drim)


def kernel(pos_triples, neg_triples, ent_re, ent_im, rel_re, rel_im):
    tri = jnp.concatenate([pos_triples, neg_triples], axis=0).astype(jnp.int32)
    tot = tri.shape[0]
    out = _complex_scores(tri[:, 0], tri[:, 1], tri[:, 2],
                          ent_re, ent_im, rel_re, rel_im, tot)
    b = pos_triples.shape[0]
    return out[:b], out[b:]


# speedup vs baseline: 1.6448x; 1.6448x over previous
import functools

import jax
import jax.numpy as jnp
from jax import lax
from jax.experimental import pallas as pl
from jax.experimental.pallas import tpu as pltpu
from jax.experimental.pallas import tpu_sc as plsc

NE = 1000000
DIM = 32
NC = 2
NS = 16
L = 16
NW = NC * NS

BLK = 512
FULL_E = (NE // 128) * 128
NBLK = FULL_E // BLK
WPT = NW // 4
TAIL = NE - FULL_E

C = 128


def _transpose_block(in_v, out_v, nrows):
    rows_lo = lax.iota(jnp.int32, L)
    rows_hi = L + lax.iota(jnp.int32, L)

    def tb(g, c2):
        colsb = jnp.full((L,), 0, jnp.int32) + g * L
        for j in range(L):
            cols = colsb + j
            r = g * 4 + j // 4
            c0 = (j % 4) * DIM
            out_v[r, pl.ds(c0, L)] = plsc.load_gather(in_v, [rows_lo, cols])
            out_v[r, pl.ds(c0 + L, L)] = plsc.load_gather(in_v, [rows_hi, cols])
        return c2

    lax.fori_loop(0, nrows // L, tb, 0)


def _detile_one(t_hbm, s_hbm, o_hbm, bslot, in_b, out_b, tl_v, si, so):
    npair = NBLK // WPT // 2 + 1
    fpb = BLK * DIM // 128

    def pair(i, carry):
        for p in range(2):
            bidx = bslot + (2 * i + p) * WPT

            @pl.when(jnp.logical_and(i > 0, bidx < NBLK))
            def _(p=p):
                pltpu.make_async_copy(
                    o_hbm.at[pl.ds(0, fpb), :], out_b[p], so[p]).wait()

            @pl.when(bidx < NBLK)
            def _(p=p, bidx=bidx):
                e0 = pl.multiple_of(bidx * BLK, 128)
                pltpu.async_copy(t_hbm.at[:, pl.ds(e0, BLK)], in_b[p], si[p])

        for p in range(2):
            bidx = bslot + (2 * i + p) * WPT

            @pl.when(bidx < NBLK)
            def _(p=p, bidx=bidx):
                e0 = pl.multiple_of(bidx * BLK, 128)
                pltpu.make_async_copy(
                    t_hbm.at[:, pl.ds(e0, BLK)], in_b[p], si[p]).wait()
                _transpose_block(in_b[p], out_b[p], BLK)
                pltpu.async_copy(
                    out_b[p], o_hbm.at[pl.ds(bidx * fpb, fpb), :], so[p])

        return carry

    lax.fori_loop(0, npair, pair, 0)
    for p in range(2):
        pltpu.make_async_copy(
            o_hbm.at[pl.ds(0, fpb), :], out_b[p], so[p]).wait()

    @pl.when(bslot == 0)
    def _():
        pltpu.sync_copy(s_hbm, tl_v)

        def row_body(e, c2):
            r = e // 4
            c0 = lax.rem(e, 4) * DIM
            for d0 in range(0, DIM, L):
                rows = d0 + lax.iota(jnp.int32, L)
                cols = jnp.full((L,), 0, jnp.int32) + e
                out_b[0][r, pl.ds(c0 + d0, L)] = plsc.load_gather(
                    tl_v, [rows, cols])
            return c2

        lax.fori_loop(0, TAIL, row_body, 0)
        tfr = TAIL * DIM // 128
        pltpu.sync_copy(out_b[0].at[pl.ds(0, tfr), :],
                        o_hbm.at[pl.ds(FULL_E * DIM // 128, tfr), :])


def _make_detile():
    mesh = plsc.VectorSubcoreMesh(core_axis_name="c", subcore_axis_name="s")
    tbl = jax.ShapeDtypeStruct((NE * DIM // 128, 128), jnp.float32)

    @functools.partial(
        pl.kernel,
        out_type=(tbl, tbl, tbl, tbl),
        mesh=mesh,
        compiler_params=pltpu.CompilerParams(
            needs_layout_passes=False, use_tc_tiling_on_sc=True),
        scratch_types=[
            pltpu.VMEM((DIM, BLK), jnp.float32),
            pltpu.VMEM((DIM, BLK), jnp.float32),
            pltpu.VMEM((BLK * DIM // 128, 128), jnp.float32),
            pltpu.VMEM((BLK * DIM // 128, 128), jnp.float32),
            pltpu.VMEM((DIM, TAIL), jnp.float32),
            pltpu.SemaphoreType.DMA,
            pltpu.SemaphoreType.DMA,
            pltpu.SemaphoreType.DMA,
            pltpu.SemaphoreType.DMA,
        ],
    )
    def k(t0, t1, t2, t3, s0, s1, s2, s3, o0, o1, o2, o3,
          in0, in1, ou0, ou1, tl_v, si0, si1, so0, so1):
        wid = lax.axis_index("s") * NC + lax.axis_index("c")
        tid = wid // WPT
        bslot = wid % WPT
        for i, (t, s, o) in enumerate(
                ((t0, s0, o0), (t1, s1, o1), (t2, s2, o2), (t3, s3, o3))):
            @pl.when(tid == i)
            def _(t=t, s=s, o=o):
                _detile_one(t, s, o, bslot, (in0, in1), (ou0, ou1), tl_v,
                            (si0, si1), (so0, so1))

    return k


def _make_score(tot):
    per_w = tot // NW
    n_chunks = per_w // C
    mesh = plsc.VectorSubcoreMesh(core_axis_name="c", subcore_axis_name="s")

    @functools.partial(
        pl.kernel,
        out_type=jax.ShapeDtypeStruct((tot,), jnp.float32),
        mesh=mesh,
        compiler_params=pltpu.CompilerParams(
            needs_layout_passes=False, use_tc_tiling_on_sc=False),
        scratch_types=[
            pltpu.VMEM((C,), jnp.int32),
            pltpu.VMEM((C,), jnp.int32),
            pltpu.VMEM((C,), jnp.int32),
            pltpu.VMEM((C, DIM), jnp.float32),
            pltpu.VMEM((C, DIM), jnp.float32),
            pltpu.VMEM((C, DIM), jnp.float32),
            pltpu.VMEM((C, DIM), jnp.float32),
            pltpu.VMEM((C, DIM), jnp.float32),
            pltpu.VMEM((C, DIM), jnp.float32),
            pltpu.VMEM((C,), jnp.float32),
            pltpu.SemaphoreType.DMA,
        ],
    )
    def k(h_hbm, r_hbm, t_hbm, ere_hbm, eim_hbm, rre_hbm, rim_hbm, out_hbm,
          h_v, r_v, t_v, hre, him, tre, tim, rre, rim, sc_v, sem):
        wid = lax.axis_index("s") * NC + lax.axis_index("c")
        w_base = wid * per_w

        def chunk_body(ci, carry):
            base = w_base + ci * C
            pltpu.sync_copy(h_hbm.at[pl.ds(base, C)], h_v)
            pltpu.sync_copy(r_hbm.at[pl.ds(base, C)], r_v)
            pltpu.sync_copy(t_hbm.at[pl.ds(base, C)], t_v)
            cps = [
                pltpu.async_copy(ere_hbm.at[h_v], hre, sem),
                pltpu.async_copy(eim_hbm.at[h_v], him, sem),
                pltpu.async_copy(ere_hbm.at[t_v], tre, sem),
                pltpu.async_copy(eim_hbm.at[t_v], tim, sem),
                pltpu.async_copy(rre_hbm.at[r_v], rre, sem),
                pltpu.async_copy(rim_hbm.at[r_v], rim, sem),
            ]
            for cp in cps:
                cp.wait()

            def group_body(g, gcarry):
                row = g * L + lax.iota(jnp.int32, L)
                acc = jnp.zeros((L,), jnp.float32)
                for d in range(DIM):
                    col = jnp.full((L,), d, jnp.int32)
                    a = plsc.load_gather(hre, [row, col])
                    b = plsc.load_gather(him, [row, col])
                    u = plsc.load_gather(tre, [row, col])
                    v = plsc.load_gather(tim, [row, col])
                    p = plsc.load_gather(rre, [row, col])
                    q = plsc.load_gather(rim, [row, col])
                    acc = acc + p * (a * u + b * v) + q * (a * v - b * u)
                sc_v[pl.ds(g * L, L)] = acc
                return gcarry

            lax.fori_loop(0, C // L, group_body, 0)
            pltpu.sync_copy(sc_v, out_hbm.at[pl.ds(base, C)])
            return carry

        lax.fori_loop(0, n_chunks, chunk_body, 0)

    return k


@functools.partial(jax.jit, static_argnames=("tot",))
def _complex_scores(h, r, t, ent_re, ent_im, rel_re, rel_im, tot):
    tables = (ent_re, ent_im, rel_re, rel_im)
    tviews = [x.T for x in tables]
    sides = [v[:, FULL_E:] for v in tviews]
    outs = _make_detile()(*tviews, *sides)
    dre, dim_, drre, drim = (jnp.reshape(x, (NE, DIM)) for x in outs)
    return _make_score(tot)(h, r, t, dre, dim_, drre, drim)


def kernel(pos_triples, neg_triples, ent_re, ent_im, rel_re, rel_im):
    tri = jnp.concatenate([pos_triples, neg_triples], axis=0).astype(jnp.int32)
    tot = tri.shape[0]
    out = _complex_scores(tri[:, 0], tri[:, 1], tri[:, 2],
                          ent_re, ent_im, rel_re, rel_im, tot)
    b = pos_triples.shape[0]
    return out[:b], out[b:]


# speedup vs baseline: 1.9868x; 1.2079x over previous
import functools

import jax
import jax.numpy as jnp
from jax import lax
from jax.experimental import pallas as pl
from jax.experimental.pallas import tpu as pltpu
from jax.experimental.pallas import tpu_sc as plsc

NE = 1000000
DIM = 32
NC = 2
NS = 16
L = 16
NW = NC * NS

BLK = 512
FULL_E = (NE // 128) * 128
NBLK = FULL_E // BLK
WPT = NW // 4
TAIL = NE - FULL_E

C = 128


def _transpose_block(in_v, out_v, nrows):
    def tb(g, c2):
        base = (g * L + lax.iota(jnp.int32, L)) * DIM
        for d in range(DIM):
            v = in_v[d, pl.ds(g * L, L)]
            plsc.store_scatter(out_v, [base + d], v)
        return c2

    lax.fori_loop(0, nrows // L, tb, 0)


def _detile_one(t_hbm, s_hbm, o_hbm, bslot, in_b, out_b, tl_v, si, so):
    npair = NBLK // WPT // 2 + 1
    fpb = BLK * DIM

    def pair(i, carry):
        for p in range(2):
            bidx = bslot + (2 * i + p) * WPT

            @pl.when(jnp.logical_and(i > 0, bidx < NBLK))
            def _(p=p):
                pltpu.make_async_copy(
                    o_hbm.at[pl.ds(0, fpb)], out_b[p], so[p]).wait()

            @pl.when(bidx < NBLK)
            def _(p=p, bidx=bidx):
                e0 = pl.multiple_of(bidx * BLK, 128)
                pltpu.async_copy(t_hbm.at[:, pl.ds(e0, BLK)], in_b[p], si[p])

        for p in range(2):
            bidx = bslot + (2 * i + p) * WPT

            @pl.when(bidx < NBLK)
            def _(p=p, bidx=bidx):
                e0 = pl.multiple_of(bidx * BLK, 128)
                pltpu.make_async_copy(
                    t_hbm.at[:, pl.ds(e0, BLK)], in_b[p], si[p]).wait()
                _transpose_block(in_b[p], out_b[p], BLK)
                pltpu.async_copy(
                    out_b[p], o_hbm.at[pl.ds(bidx * fpb, fpb)], so[p])

        return carry

    lax.fori_loop(0, npair, pair, 0)
    for p in range(2):
        pltpu.make_async_copy(
            o_hbm.at[pl.ds(0, fpb)], out_b[p], so[p]).wait()

    @pl.when(bslot == 0)
    def _():
        pltpu.sync_copy(s_hbm, tl_v)

        def tg(g, c2):
            base = (g * L + lax.iota(jnp.int32, L)) * DIM
            for d in range(DIM):
                v = tl_v[d, pl.ds(g * L, L)]
                plsc.store_scatter(out_b[0], [base + d], v)
            return c2

        lax.fori_loop(0, TAIL // L, tg, 0)
        pltpu.sync_copy(out_b[0].at[pl.ds(0, TAIL * DIM)],
                        o_hbm.at[pl.ds(FULL_E * DIM, TAIL * DIM)])


def _make_detile():
    mesh = plsc.VectorSubcoreMesh(core_axis_name="c", subcore_axis_name="s")
    tbl = jax.ShapeDtypeStruct((NE * DIM,), jnp.float32)

    @functools.partial(
        pl.kernel,
        out_type=(tbl, tbl, tbl, tbl),
        mesh=mesh,
        compiler_params=pltpu.CompilerParams(
            needs_layout_passes=False, use_tc_tiling_on_sc=True),
        scratch_types=[
            pltpu.VMEM((DIM, BLK), jnp.float32),
            pltpu.VMEM((DIM, BLK), jnp.float32),
            pltpu.VMEM((BLK * DIM,), jnp.float32),
            pltpu.VMEM((BLK * DIM,), jnp.float32),
            pltpu.VMEM((DIM, TAIL), jnp.float32),
            pltpu.SemaphoreType.DMA,
            pltpu.SemaphoreType.DMA,
            pltpu.SemaphoreType.DMA,
            pltpu.SemaphoreType.DMA,
        ],
    )
    def k(t0, t1, t2, t3, s0, s1, s2, s3, o0, o1, o2, o3,
          in0, in1, ou0, ou1, tl_v, si0, si1, so0, so1):
        wid = lax.axis_index("s") * NC + lax.axis_index("c")
        tid = wid // WPT
        bslot = wid % WPT
        for i, (t, s, o) in enumerate(
                ((t0, s0, o0), (t1, s1, o1), (t2, s2, o2), (t3, s3, o3))):
            @pl.when(tid == i)
            def _(t=t, s=s, o=o):
                _detile_one(t, s, o, bslot, (in0, in1), (ou0, ou1), tl_v,
                            (si0, si1), (so0, so1))

    return k


def _make_score(tot):
    per_w = tot // NW
    n_chunks = per_w // C
    mesh = plsc.VectorSubcoreMesh(core_axis_name="c", subcore_axis_name="s")

    @functools.partial(
        pl.kernel,
        out_type=jax.ShapeDtypeStruct((tot,), jnp.float32),
        mesh=mesh,
        compiler_params=pltpu.CompilerParams(
            needs_layout_passes=False, use_tc_tiling_on_sc=False),
        scratch_types=[
            pltpu.VMEM((C,), jnp.int32),
            pltpu.VMEM((C,), jnp.int32),
            pltpu.VMEM((C,), jnp.int32),
            pltpu.VMEM((C, DIM), jnp.float32),
            pltpu.VMEM((C, DIM), jnp.float32),
            pltpu.VMEM((C, DIM), jnp.float32),
            pltpu.VMEM((C, DIM), jnp.float32),
            pltpu.VMEM((C, DIM), jnp.float32),
            pltpu.VMEM((C, DIM), jnp.float32),
            pltpu.VMEM((C,), jnp.float32),
            pltpu.SemaphoreType.DMA,
        ],
    )
    def k(h_hbm, r_hbm, t_hbm, ere_hbm, eim_hbm, rre_hbm, rim_hbm, out_hbm,
          h_v, r_v, t_v, hre, him, tre, tim, rre, rim, sc_v, sem):
        wid = lax.axis_index("s") * NC + lax.axis_index("c")
        w_base = wid * per_w

        def chunk_body(ci, carry):
            base = w_base + ci * C
            pltpu.sync_copy(h_hbm.at[pl.ds(base, C)], h_v)
            pltpu.sync_copy(r_hbm.at[pl.ds(base, C)], r_v)
            pltpu.sync_copy(t_hbm.at[pl.ds(base, C)], t_v)
            cps = [
                pltpu.async_copy(ere_hbm.at[h_v], hre, sem),
                pltpu.async_copy(eim_hbm.at[h_v], him, sem),
                pltpu.async_copy(ere_hbm.at[t_v], tre, sem),
                pltpu.async_copy(eim_hbm.at[t_v], tim, sem),
                pltpu.async_copy(rre_hbm.at[r_v], rre, sem),
                pltpu.async_copy(rim_hbm.at[r_v], rim, sem),
            ]
            for cp in cps:
                cp.wait()

            def group_body(g, gcarry):
                row = g * L + lax.iota(jnp.int32, L)
                acc = jnp.zeros((L,), jnp.float32)
                for d in range(DIM):
                    col = jnp.full((L,), d, jnp.int32)
                    a = plsc.load_gather(hre, [row, col])
                    b = plsc.load_gather(him, [row, col])
                    u = plsc.load_gather(tre, [row, col])
                    v = plsc.load_gather(tim, [row, col])
                    p = plsc.load_gather(rre, [row, col])
                    q = plsc.load_gather(rim, [row, col])
                    acc = acc + p * (a * u + b * v) + q * (a * v - b * u)
                sc_v[pl.ds(g * L, L)] = acc
                return gcarry

            lax.fori_loop(0, C // L, group_body, 0)
            pltpu.sync_copy(sc_v, out_hbm.at[pl.ds(base, C)])
            return carry

        lax.fori_loop(0, n_chunks, chunk_body, 0)

    return k


@functools.partial(jax.jit, static_argnames=("tot",))
def _complex_scores(h, r, t, ent_re, ent_im, rel_re, rel_im, tot):
    tables = (ent_re, ent_im, rel_re, rel_im)
    tviews = [x.T for x in tables]
    sides = [v[:, FULL_E:] for v in tviews]
    outs = _make_detile()(*tviews, *sides)
    dre, dim_, drre, drim = (jnp.reshape(x, (NE, DIM)) for x in outs)
    return _make_score(tot)(h, r, t, dre, dim_, drre, drim)


def kernel(pos_triples, neg_triples, ent_re, ent_im, rel_re, rel_im):
    tri = jnp.concatenate([pos_triples, neg_triples], axis=0).astype(jnp.int32)
    tot = tri.shape[0]
    out = _complex_scores(tri[:, 0], tri[:, 1], tri[:, 2],
                          ent_re, ent_im, rel_re, rel_im, tot)
    b = pos_triples.shape[0]
    return out[:b], out[b:]


# speedup vs baseline: 3.0850x; 1.5527x over previous
import functools

import jax
import jax.numpy as jnp
from jax import lax
from jax.experimental import pallas as pl
from jax.experimental.pallas import tpu as pltpu
from jax.experimental.pallas import tpu_sc as plsc

DIM = 32
NC = 2
NS = 16
L = 16
NW = NC * NS
C = 128


@functools.partial(jax.jit, static_argnames=("tot",))
def _sc_scores(h, r, t, ent_re, ent_im, rel_re, rel_im, tot):
    per_w = tot // NW
    n_chunks = per_w // C
    mesh = plsc.VectorSubcoreMesh(core_axis_name="c", subcore_axis_name="s")

    @functools.partial(
        pl.kernel,
        out_type=jax.ShapeDtypeStruct((tot,), jnp.float32),
        mesh=mesh,
        compiler_params=pltpu.CompilerParams(
            needs_layout_passes=False, use_tc_tiling_on_sc=False),
        scratch_types=[
            pltpu.VMEM((C,), jnp.int32),
            pltpu.VMEM((C,), jnp.int32),
            pltpu.VMEM((C,), jnp.int32),
            pltpu.VMEM((C, DIM), jnp.float32),
            pltpu.VMEM((C, DIM), jnp.float32),
            pltpu.VMEM((C, DIM), jnp.float32),
            pltpu.VMEM((C, DIM), jnp.float32),
            pltpu.VMEM((C, DIM), jnp.float32),
            pltpu.VMEM((C, DIM), jnp.float32),
            pltpu.VMEM((C,), jnp.float32),
            pltpu.SemaphoreType.DMA,
        ],
    )
    def k(h_hbm, r_hbm, t_hbm, ere_hbm, eim_hbm, rre_hbm, rim_hbm, out_hbm,
          h_v, r_v, t_v, hre, him, tre, tim, rre, rim, sc_v, sem):
        wid = lax.axis_index("s") * NC + lax.axis_index("c")
        w_base = wid * per_w

        def chunk_body(ci, carry):
            base = w_base + ci * C
            pltpu.sync_copy(h_hbm.at[pl.ds(base, C)], h_v)
            pltpu.sync_copy(r_hbm.at[pl.ds(base, C)], r_v)
            pltpu.sync_copy(t_hbm.at[pl.ds(base, C)], t_v)
            cps = [
                pltpu.async_copy(ere_hbm.at[h_v], hre, sem),
                pltpu.async_copy(eim_hbm.at[h_v], him, sem),
                pltpu.async_copy(ere_hbm.at[t_v], tre, sem),
                pltpu.async_copy(eim_hbm.at[t_v], tim, sem),
                pltpu.async_copy(rre_hbm.at[r_v], rre, sem),
                pltpu.async_copy(rim_hbm.at[r_v], rim, sem),
            ]
            for cp in cps:
                cp.wait()

            def group_body(g, gcarry):
                row = g * L + lax.iota(jnp.int32, L)
                acc = jnp.zeros((L,), jnp.float32)
                for d in range(DIM):
                    col = jnp.full((L,), d, jnp.int32)
                    a = plsc.load_gather(hre, [row, col])
                    b = plsc.load_gather(him, [row, col])
                    u = plsc.load_gather(tre, [row, col])
                    v = plsc.load_gather(tim, [row, col])
                    p = plsc.load_gather(rre, [row, col])
                    q = plsc.load_gather(rim, [row, col])
                    acc = acc + p * (a * u + b * v) + q * (a * v - b * u)
                sc_v[pl.ds(g * L, L)] = acc
                return gcarry

            lax.fori_loop(0, C // L, group_body, 0)
            pltpu.sync_copy(sc_v, out_hbm.at[pl.ds(base, C)])
            return carry

        lax.fori_loop(0, n_chunks, chunk_body, 0)

    return k(h, r, t, ent_re, ent_im, rel_re, rel_im)


def kernel(pos_triples, neg_triples, ent_re, ent_im, rel_re, rel_im):
    tri = jnp.concatenate([pos_triples, neg_triples], axis=0).astype(jnp.int32)
    tot = tri.shape[0]
    out = _sc_scores(tri[:, 0], tri[:, 1], tri[:, 2],
                     ent_re, ent_im, rel_re, rel_im, tot)
    b = pos_triples.shape[0]
    return out[:b], out[b:]
